# baseline (speedup 1.0000x reference)
import functools
from functools import partial

import jax
import jax.numpy as jnp
from jax import lax
from jax.experimental import pallas as pl
from jax.experimental.pallas import tpu as pltpu
from jax.experimental.pallas import tpu_sc as plsc

_SC_CORES = 2
_SC_SUBCORES = 16
_SC_WORKERS = _SC_CORES * _SC_SUBCORES
_GCHUNK = 80
_GROWS = 4000
_GROWS_PW = _GROWS // _SC_WORKERS


def _sc_gather_body(a_hbm, b_hbm, dix_hbm, six_hbm, ga_hbm, gb_hbm,
                    dix_v, six_v, bufa, bufb, gsem):
    wid = lax.axis_index("s") * _SC_CORES + lax.axis_index("c")
    row0 = wid * _GROWS_PW
    pltpu.sync_copy(dix_hbm.at[wid], dix_v)
    pltpu.sync_copy(six_hbm.at[wid], six_v)

    def body(k, _):
        c1 = pltpu.async_copy(a_hbm.at[dix_v.at[k]], bufa, gsem)
        c2 = pltpu.async_copy(b_hbm.at[six_v.at[k]], bufb, gsem)
        c1.wait()
        c2.wait()
        e0 = (row0 + k) * _GCHUNK
        pltpu.sync_copy(bufa, ga_hbm.at[pl.ds(e0, _GCHUNK)])
        pltpu.sync_copy(bufb, gb_hbm.at[pl.ds(e0, _GCHUNK)])
        return 0

    lax.fori_loop(0, _GROWS_PW, body, 0)


def _sc_gather(a, b, dix2d, six2d):
    e = _GROWS * _GCHUNK
    h = a.shape[1]
    mesh = plsc.VectorSubcoreMesh(core_axis_name="c", subcore_axis_name="s")
    fn = functools.partial(
        pl.kernel,
        mesh=mesh,
        compiler_params=pltpu.CompilerParams(use_tc_tiling_on_sc=False, needs_layout_passes=False),
        out_type=[
            jax.ShapeDtypeStruct((e, h), jnp.float32),
            jax.ShapeDtypeStruct((e, h), jnp.float32),
        ],
        scratch_types=[
            pltpu.VMEM((_GROWS_PW, _GCHUNK), jnp.int32),
            pltpu.VMEM((_GROWS_PW, _GCHUNK), jnp.int32),
            pltpu.VMEM((_GCHUNK, h), jnp.float32),
            pltpu.VMEM((_GCHUNK, h), jnp.float32),
            pltpu.SemaphoreType.DMA,
        ],
    )(_sc_gather_body)
    return fn(a, b, dix2d, six2d)

_BN_EPS = 1e-5
_N_NODES = 10000
_NODE_BLK = 1000
_EDGE_BLK = 2000


_NPW = 313
_SCAP = 16384
_MCHUNK = 80


def _zero_f32_2d(ref, nrows, ncols16):
    z = jnp.zeros((16,), jnp.float32)

    def body(r, _):
        for c in range(ncols16):
            ref[r, pl.ds(c * 16, 16)] = z
        return 0

    lax.fori_loop(0, nrows, body, 0)


def _apply_max(r3_hbm, eidbuf, dlbuf, tbl, gb, sem, cnt):
    nch = (cnt + _MCHUNK - 1) // _MCHUNK

    def fire(k, slot):
        return pltpu.async_copy(
            r3_hbm.at[eidbuf.at[pl.ds(k * _MCHUNK, _MCHUNK)]], gb.at[slot], sem)

    @pl.when(nch > 0)
    def _():
        fire(0, 0)

        def chunk(k, _):
            @pl.when(k + 1 < nch)
            def _():
                fire(k + 1, (k + 1) % 2)

            pltpu.make_async_copy(
                r3_hbm.at[eidbuf.at[pl.ds(k * _MCHUNK, _MCHUNK)]],
                gb.at[k % 2], sem).wait()
            rem = jnp.minimum(cnt - k * _MCHUNK, _MCHUNK)
            slot = k % 2

            def row(i, _):
                dl = dlbuf[pl.ds(k * _MCHUNK + i, 16)][0]
                for c in range(4):
                    sl = pl.ds(c * 16, 16)
                    tbl[dl, sl] = jnp.maximum(tbl[dl, sl], gb[slot, i, sl])
                return 0

            lax.fori_loop(0, rem, row, 0)
            return 0

        lax.fori_loop(0, nch, chunk, 0)


def _write_table(tbl, out_hbm, wid, lo):
    @pl.when(wid < _SC_WORKERS - 1)
    def _():
        pltpu.sync_copy(tbl.at[pl.ds(0, _NPW)], out_hbm.at[pl.ds(lo, _NPW)])

    @pl.when(wid == _SC_WORKERS - 1)
    def _():
        last = 10000 - (_SC_WORKERS - 1) * _NPW
        pltpu.sync_copy(tbl.at[pl.ds(0, last)], out_hbm.at[pl.ds(lo, last)])


def _sc_scatmax_scan_body(r3_hbm, dix_hbm, out_hbm, eidl_hbm, dll_hbm, cnt_hbm,
                          dstbuf, mbuf, eidbuf, dlbuf, cbuf, tbl, gb, sem):
    wid = lax.axis_index("s") * _SC_CORES + lax.axis_index("c")
    lo = wid * _NPW
    hi = jnp.minimum(lo + _NPW, 10000)
    _zero_f32_2d(tbl, _NPW + 7, 4)
    zi = jnp.zeros((16,), jnp.int32)

    def zb(i, _):
        eidbuf[pl.ds(i * 16, 16)] = zi
        return 0

    lax.fori_loop(0, _SCAP // 16, zb, 0)
    cbuf[pl.ds(0, 16)] = zi

    lane = lax.iota(jnp.int32, 16)
    zero16 = jnp.zeros((16,), jnp.int32)

    def slice_body(s, _):
        pltpu.sync_copy(dix_hbm.at[s], dstbuf)

        def row_body(r, _):
            for g in range(_GCHUNK // 16):
                d = dstbuf[r, pl.ds(g * 16, 16)]
                dl = d - lo
                m01 = jnp.minimum(jnp.maximum(dl + 1, 0), 1) * jnp.minimum(
                    jnp.maximum(hi - d, 0), 1)
                base = s * 10000 + r * _GCHUNK + g * 16
                packed = dl * 524288 + (base + lane)
                sp = plsc.sort_key_val(1 - m01, packed)[1]
                cnt = cbuf[pl.ds(0, 16)][0]
                mbuf[pl.ds(cnt, 16)] = sp
                plsc.addupdate_scatter(cbuf, [zero16], m01)
            return 0

        return lax.fori_loop(0, _GROWS_PW, row_body, 0)

    lax.fori_loop(0, _SC_WORKERS, slice_body, 0)

    cnt = cbuf[pl.ds(0, 16)][0]
    nug = (cnt + 15) // 16
    cv = jnp.full((16,), cnt, jnp.int32)

    def unp(g, _):
        pv = mbuf[pl.ds(g * 16, 16)]
        sl = jnp.full((16,), g * 16, jnp.int32) + lane
        valid = jnp.minimum(jnp.maximum(cv - sl, 0), 1)
        eidbuf[pl.ds(g * 16, 16)] = (pv & 524287) * valid
        dlbuf[pl.ds(g * 16, 16)] = (pv >> 19) * valid
        return 0

    lax.fori_loop(0, nug, unp, 0)

    pltpu.sync_copy(eidbuf, eidl_hbm.at[wid])
    pltpu.sync_copy(dlbuf, dll_hbm.at[wid])
    pltpu.sync_copy(cbuf, cnt_hbm.at[wid])

    _apply_max(r3_hbm, eidbuf, dlbuf, tbl, gb, sem, cnt)
    _write_table(tbl, out_hbm, wid, lo)


def _sc_scatmax_reuse_body(r3_hbm, eidl_hbm, dll_hbm, cnt_hbm, out_hbm,
                           eidbuf, dlbuf, cbuf, tbl, gb, sem):
    wid = lax.axis_index("s") * _SC_CORES + lax.axis_index("c")
    lo = wid * _NPW
    _zero_f32_2d(tbl, _NPW + 7, 4)
    pltpu.sync_copy(eidl_hbm.at[wid], eidbuf)
    pltpu.sync_copy(dll_hbm.at[wid], dlbuf)
    pltpu.sync_copy(cnt_hbm.at[wid], cbuf)
    cnt = cbuf[pl.ds(0, 16)][0]
    _apply_max(r3_hbm, eidbuf, dlbuf, tbl, gb, sem, cnt)
    _write_table(tbl, out_hbm, wid, lo)


def _scat_scratch():
    return [
        pltpu.VMEM((_SCAP,), jnp.int32),
        pltpu.VMEM((_SCAP,), jnp.int32),
        pltpu.VMEM((16,), jnp.int32),
        pltpu.VMEM((_NPW + 7, 64), jnp.float32),
        pltpu.VMEM((2, _MCHUNK, 64), jnp.float32),
        pltpu.SemaphoreType.DMA,
    ]


def _sc_scatter_scan(r3, dix3d):
    mesh = plsc.VectorSubcoreMesh(core_axis_name="c", subcore_axis_name="s")
    fn = functools.partial(
        pl.kernel,
        mesh=mesh,
        compiler_params=pltpu.CompilerParams(use_tc_tiling_on_sc=False, needs_layout_passes=False),
        out_type=[
            jax.ShapeDtypeStruct((10000, 64), jnp.float32),
            jax.ShapeDtypeStruct((_SC_WORKERS, _SCAP), jnp.int32),
            jax.ShapeDtypeStruct((_SC_WORKERS, _SCAP), jnp.int32),
            jax.ShapeDtypeStruct((_SC_WORKERS, 16), jnp.int32),
        ],
        scratch_types=[pltpu.VMEM((_GROWS_PW, _GCHUNK), jnp.int32),
                       pltpu.VMEM((_SCAP,), jnp.int32)] + _scat_scratch(),
    )(_sc_scatmax_scan_body)
    return fn(r3, dix3d)


def _sc_scatter_reuse(r3, eidl, dll, cnts):
    mesh = plsc.VectorSubcoreMesh(core_axis_name="c", subcore_axis_name="s")
    fn = functools.partial(
        pl.kernel,
        mesh=mesh,
        compiler_params=pltpu.CompilerParams(use_tc_tiling_on_sc=False, needs_layout_passes=False),
        out_type=jax.ShapeDtypeStruct((10000, 64), jnp.float32),
        scratch_types=_scat_scratch(),
    )(_sc_scatmax_reuse_body)
    return fn(r3, eidl, dll, cnts)


def _precompute_body(pos_ref, feat_ref, wa_ref, wb_ref, wc_ref, b_ref, a_out, b_out):
    pos = pos_ref[...]
    feat = feat_ref[...]
    a = jnp.dot(pos, wa_ref[...], preferred_element_type=jnp.float32)
    a = a + jnp.dot(feat, wc_ref[...], preferred_element_type=jnp.float32)
    a_out[...] = a + b_ref[...]
    b_out[...] = jnp.dot(pos, wb_ref[...], preferred_element_type=jnp.float32)


def _node_precompute(pos, feat, w1, b1):
    n, f = feat.shape
    h = w1.shape[1]
    wa = w1[0:3] - w1[3:6]
    wb = w1[3:6]
    wc = w1[6:]
    grid = n // _NODE_BLK
    return pl.pallas_call(
        _precompute_body,
        grid=(grid,),
        in_specs=[
            pl.BlockSpec((_NODE_BLK, 3), lambda i: (i, 0)),
            pl.BlockSpec((_NODE_BLK, f), lambda i: (i, 0)),
            pl.BlockSpec((3, h), lambda i: (0, 0)),
            pl.BlockSpec((3, h), lambda i: (0, 0)),
            pl.BlockSpec((f, h), lambda i: (0, 0)),
            pl.BlockSpec((1, h), lambda i: (0, 0)),
        ],
        out_specs=[
            pl.BlockSpec((_NODE_BLK, h), lambda i: (i, 0)),
            pl.BlockSpec((_NODE_BLK, h), lambda i: (i, 0)),
        ],
        out_shape=[
            jax.ShapeDtypeStruct((n, h), jnp.float32),
            jax.ShapeDtypeStruct((n, h), jnp.float32),
        ],
    )(pos, feat, wa, wb, wc, b1.reshape(1, h))


def _edge_mlp_body(ga_ref, gb_ref, w2_ref, b2_ref, w3_ref, b3_ref, s3_ref, t3_ref, out_ref):
    h1 = jnp.maximum(ga_ref[...] + gb_ref[...], 0.0)
    u2 = jnp.dot(h1, w2_ref[...], preferred_element_type=jnp.float32) + b2_ref[...]
    h2 = jnp.maximum(u2, 0.0)
    u3 = jnp.dot(h2, w3_ref[...], preferred_element_type=jnp.float32) + b3_ref[...]
    h3 = s3_ref[...] * jnp.maximum(u3, 0.0) + t3_ref[...]
    out_ref[...] = jnp.maximum(h3, 0.0)


def _edge_mlp(ga, gb, w2f, b2f, w3f, b3f, s3, t3):
    e, h = ga.shape
    grid = e // _EDGE_BLK
    return pl.pallas_call(
        _edge_mlp_body,
        grid=(grid,),
        in_specs=[
            pl.BlockSpec((_EDGE_BLK, h), lambda i: (i, 0)),
            pl.BlockSpec((_EDGE_BLK, h), lambda i: (i, 0)),
            pl.BlockSpec((h, h), lambda i: (0, 0)),
            pl.BlockSpec((1, h), lambda i: (0, 0)),
            pl.BlockSpec((h, h), lambda i: (0, 0)),
            pl.BlockSpec((1, h), lambda i: (0, 0)),
            pl.BlockSpec((1, h), lambda i: (0, 0)),
            pl.BlockSpec((1, h), lambda i: (0, 0)),
        ],
        out_specs=pl.BlockSpec((_EDGE_BLK, h), lambda i: (i, 0)),
        out_shape=jax.ShapeDtypeStruct((e, h), jnp.float32),
    )(ga, gb, w2f, b2f.reshape(1, h), w3f, b3f.reshape(1, h),
      s3.reshape(1, h), t3.reshape(1, h))


def _head_body(m1_ref, m2_ref, lw1_ref, lb1_ref, lw2_ref, lb2_ref, out_ref):
    n = m1_ref.shape[0]
    mean1 = jnp.sum(m1_ref[...], axis=0, keepdims=True) * (1.0 / n)
    mean2 = jnp.sum(m2_ref[...], axis=0, keepdims=True) * (1.0 / n)
    j = jnp.concatenate([mean1, mean2], axis=1)
    z = jnp.maximum(jnp.dot(j, lw1_ref[...], preferred_element_type=jnp.float32)
                    + lb1_ref[...], 0.0)
    logits = jnp.dot(z, lw2_ref[...], preferred_element_type=jnp.float32) + lb2_ref[...]
    mx = jnp.max(logits, axis=1, keepdims=True)
    lse = jnp.log(jnp.sum(jnp.exp(logits - mx), axis=1, keepdims=True)) + mx
    out_ref[...] = logits - lse


def _head(m1, m2, lw1, lb1, lw2, lb2):
    n, h = m1.shape
    ncls = lw2.shape[1]
    return pl.pallas_call(
        _head_body,
        out_shape=jax.ShapeDtypeStruct((1, ncls), jnp.float32),
    )(m1, m2, lw1, lb1.reshape(1, h), lw2, lb2.reshape(1, ncls))


def _fold_bn(params):
    c = 1.0 / jnp.sqrt(1.0 + _BN_EPS)
    w1, b1, g1, be1 = params[0:4]
    w2, b2, g2, be2 = params[4:8]
    w3, b3, g3, be3 = params[8:12]
    s1, t1 = g1 * c, be1
    s2, t2 = g2 * c, be2
    s3, t3 = g3 * c, be3
    w2f = s1[:, None] * w2
    b2f = t1 @ w2 + b2
    w3f = s2[:, None] * w3
    b3f = t2 @ w3 + b3
    return w1, b1, w2f, b2f, w3f, b3f, s3, t3


def _conv_edge_mlp(pos, feat, dix2d, six2d, params):
    w1, b1, w2f, b2f, w3f, b3f, s3, t3 = _fold_bn(params)
    a, b = _node_precompute(pos, feat, w1, b1)
    ga, gb = _sc_gather(a, b, dix2d, six2d)
    return _edge_mlp(ga, gb, w2f, b2f, w3f, b3f, s3, t3)


def kernel(x, pos, edge_index, batch, p1, p2, lin):
    dst = edge_index[1]
    src = edge_index[0]
    dix2d = dst.reshape(_SC_WORKERS, _GROWS_PW, _GCHUNK)
    six2d = src.reshape(_SC_WORKERS, _GROWS_PW, _GCHUNK)
    r3a = _conv_edge_mlp(pos, x, dix2d, six2d, p1)
    h1, eidl, dll, cnts = _sc_scatter_scan(r3a, dix2d)
    r3b = _conv_edge_mlp(pos, h1, dix2d, six2d, p2)
    h2 = _sc_scatter_reuse(r3b, eidl, dll, cnts)
    lw1, lb1, lw2, lb2 = lin
    return _head(h1, h2, lw1, lb1, lw2, lb2)
